# baseline (speedup 1.0000x reference)
import jax
import jax.numpy as jnp
from jax import lax
from jax.experimental import pallas as pl
from jax.experimental.pallas import tpu as pltpu
from jax.experimental.pallas import tpu_sc as plsc

NROWS = 128
NCOLS = 32768
K = 64
NC, NS, L = 2, 16, 16
NW = NC * NS
ROWS_PER_W = NROWS // NW
NVREG = NCOLS // L
NBINS = 256
HSTRIDE = NBINS + 1
HSIZE = HSTRIDE * L
NHIST = 8
NCHUNK = 8
CWORDS = NCOLS // NCHUNK
CVREG = CWORDS // L


def _tr(k):
    return jnp.where(k < 0, k ^ jnp.int32(0x7FFFFFFF), k)


def _body(x_hbm, out_hbm, row_v, idxs_v, hist_v, didx_v, ostage_v, sems,
          sem_out):
    wid = lax.axis_index("s") * NC + lax.axis_index("c")
    row0 = wid * ROWS_PER_W
    lanes = lax.iota(jnp.int32, L)
    ones = jnp.ones((L,), jnp.int32)
    zeros = jnp.zeros((L,), jnp.int32)
    tmask = lanes >= 0
    lane0 = lanes == 0
    lanebins = lanes * HSTRIDE

    def zero_hist(ncopies):
        @plsc.parallel_loop(0, (HSTRIDE * ncopies * L) // L, unroll=4)
        def _zb(i):
            hist_v[pl.ds(i * L, L)] = zeros

    def pass1_chunk(c0):
        @plsc.parallel_loop(0, CVREG // NHIST, unroll=4)
        def _p1(j):
            for t in range(NHIST):
                base = c0 + (j * NHIST + t) * L
                ikey = _tr(row_v[pl.ds(base, L)])
                digit = ((ikey >> 24) & 0xFF) ^ 0x80
                plsc.addupdate_scatter(
                    hist_v, [digit + (lanebins + t * HSIZE)], ones,
                    mask=tmask)

    def gather_keys(iv, valid):
        return _tr(plsc.load_gather(row_v, [iv], mask=valid))

    def hist_pass(M, shift, hi):
        zero_hist(1)
        def hb(j, _):
            base = j * L
            valid = (base + lanes) < M
            kv = gather_keys(idxs_v[pl.ds(base, L)], valid)
            digit = jnp.where(kv > hi, 255, (kv >> shift) & 0xFF)
            plsc.addupdate_scatter(hist_v, [digit + lanebins], ones,
                                   mask=valid)
            return 0
        lax.fori_loop(0, (M + L - 1) // L, hb, 0)

    def find_bin(ncopies):
        def fbv(i, carry):
            b, found, sab = carry
            v = L - 1 - i
            def lr(l, acc):
                a = acc
                for c in range(ncopies):
                    a = a + hist_v[pl.ds(l * HSTRIDE + c * HSIZE + v * L, L)]
                return a
            t_v = lax.fori_loop(0, L, lr, zeros, unroll=4)
            rc_v = lax.rev(plsc.cumsum(lax.rev(t_v, (0,))), (0,))
            need = K - sab
            cnt = jnp.sum((rc_v >= need).astype(jnp.int32))
            hit = cnt > 0
            b = jnp.where(found | (~hit), b, v * L + cnt - 1)
            return b, found | hit, sab + jnp.max(rc_v)
        b, _, _ = lax.fori_loop(
            0, L, fbv, (jnp.int32(-1), jnp.bool_(False), jnp.int32(0)))
        return b

    def split1(tp):
        @plsc.parallel_loop(0, NVREG, unroll=8, carry=zeros)
        def _sb(j, boff):
            base = j * L
            kv = _tr(row_v[pl.ds(base, L)])
            m = kv >= tp
            c = plsc.cumsum(m.astype(jnp.int32))
            plsc.store_scatter(idxs_v, [boff + c - 1], base + lanes, mask=m)
            return boff + plsc.all_reduce_population_count(m)
        return _sb

    def split(M, tp):
        def sb(j, boff):
            base = j * L
            valid = (base + lanes) < M
            iv = idxs_v[pl.ds(base, L)]
            kv = gather_keys(iv, valid)
            m = (kv >= tp) & valid
            c = plsc.cumsum(m.astype(jnp.int32))
            plsc.store_scatter(idxs_v, [boff + c - 1], iv, mask=m)
            return boff + plsc.all_reduce_population_count(m)
        return lax.fori_loop(0, (M + L - 1) // L, sb, zeros)

    def chunk_copy(row, c):
        return pltpu.make_async_copy(
            x_hbm.at[row, pl.ds(c * CWORDS, CWORDS)],
            row_v.at[pl.ds(c * CWORDS, CWORDS)], sems[c])

    def row_body(r, _):
        row = row0 + r
        for c in range(NCHUNK):
            chunk_copy(row, c).start()
        zero_hist(NHIST)
        for c in range(NCHUNK):
            chunk_copy(row, c).wait()
            pass1_chunk(c * CWORDS)

        b1 = find_bin(NHIST)
        tp = ((b1 ^ 0x80) & 0xFF) << 24
        boff = split1(tp)

        def level(lvl, carry):
            tp, M = carry
            shift = 16 - lvl * 8
            hi = tp | (lax.shift_left(jnp.int32(1), shift + 8) - 1)
            hist_pass(M, shift, hi)
            b = find_bin(1)
            tp = tp | lax.shift_left(b, shift)
            boff = split(M, tp)
            return tp, jnp.max(boff)
        tp, M = lax.fori_loop(0, 3, level, (tp, jnp.max(boff)))

        def fb(j, carry):
            doff, boff = carry
            base = j * L
            valid = (base + lanes) < M
            iv = idxs_v[pl.ds(base, L)]
            kv = gather_keys(iv, valid)
            m_gt = (kv > tp) & valid
            m_eq = (kv == tp) & valid
            c_gt = plsc.cumsum(m_gt.astype(jnp.int32))
            plsc.store_scatter(didx_v, [doff + c_gt - 1], iv, mask=m_gt)
            c_eq = plsc.cumsum(m_eq.astype(jnp.int32))
            plsc.store_scatter(idxs_v, [boff + c_eq - 1], iv, mask=m_eq)
            doff = doff + plsc.all_reduce_population_count(m_gt)
            boff = boff + plsc.all_reduce_population_count(m_eq)
            return doff, boff
        doff, _ = lax.fori_loop(0, (M + L - 1) // L, fb, (zeros, zeros))
        kneed = K - jnp.max(doff)
        doffs = jnp.max(doff)

        def ap(j, _):
            pos = j * L + lanes
            m = pos < kneed
            iv = idxs_v[pl.ds(j * L, L)]
            plsc.store_scatter(didx_v, [doffs + pos], iv, mask=m)
            return 0
        lax.fori_loop(0, (kneed + L - 1) // L, ap, 0)

        Is = [didx_v[pl.ds(a * L, L)] for a in range(K // L)]
        Ks = [gather_keys(iv, tmask) for iv in Is]
        rsplat = jnp.full((L,), r, jnp.int32)

        @plsc.parallel_loop(0, K, unroll=4)
        def _rk(c):
            csplat = jnp.full((L,), c, jnp.int32)
            ic = plsc.load_gather(didx_v, [csplat])
            kc = gather_keys(ic, tmask)
            rank = zeros
            for a in range(K // L):
                m = (Ks[a] > kc) | ((Ks[a] == kc) & (Is[a] < ic))
                rank = rank + plsc.all_reduce_population_count(m)
            plsc.store_scatter(ostage_v, [rsplat, rank], ic, mask=lane0)
        return 0

    lax.fori_loop(0, ROWS_PER_W, row_body, 0)
    pltpu.make_async_copy(ostage_v, out_hbm.at[pl.ds(row0, ROWS_PER_W)],
                          sem_out).start()
    pltpu.make_async_copy(ostage_v, out_hbm.at[pl.ds(row0, ROWS_PER_W)],
                          sem_out).wait()


@jax.jit
def kernel(input_tensor):
    x_i32 = lax.bitcast_convert_type(input_tensor, jnp.int32)
    mesh = plsc.VectorSubcoreMesh(core_axis_name="c", subcore_axis_name="s",
                                  num_cores=NC, num_subcores=NS)
    f = pl.kernel(
        _body,
        out_type=jax.ShapeDtypeStruct((NROWS, K), jnp.int32),
        mesh=mesh,
        scratch_types=[
            pltpu.VMEM((NCOLS,), jnp.int32),
            pltpu.VMEM((NCOLS,), jnp.int32),
            pltpu.VMEM((HSIZE * NHIST,), jnp.int32),
            pltpu.VMEM((K,), jnp.int32),
            pltpu.VMEM((ROWS_PER_W, K), jnp.int32),
            [pltpu.SemaphoreType.DMA] * NCHUNK,
            pltpu.SemaphoreType.DMA,
        ],
        compiler_params=pltpu.CompilerParams(
            needs_layout_passes=False,
            use_tc_tiling_on_sc=True,
        ),
    )
    return f(x_i32)


# speedup vs baseline: 1.1746x; 1.1746x over previous
import jax
import jax.numpy as jnp
from jax import lax
from jax.experimental import pallas as pl
from jax.experimental.pallas import tpu as pltpu
from jax.experimental.pallas import tpu_sc as plsc

NROWS = 128
NCOLS = 32768
K = 64
NC, NS, L = 2, 16, 16
NW = NC * NS
ROWS_PER_W = NROWS // NW
NVREG = NCOLS // L
NBINS = 256
HSTRIDE = NBINS + 1
HSIZE = HSTRIDE * L
NHIST = 4
NCHUNK = 8
CWORDS = NCOLS // NCHUNK
CVREG = CWORDS // L


def _tr(k):
    return jnp.where(k < 0, k ^ jnp.int32(0x7FFFFFFF), k)


def _body(x_hbm, out_hbm, row_v, idxs_v, hist_v, didx_v, ostage_v, sems,
          sem_out):
    wid = lax.axis_index("s") * NC + lax.axis_index("c")
    row0 = wid * ROWS_PER_W
    lanes = lax.iota(jnp.int32, L)
    ones = jnp.ones((L,), jnp.int32)
    zeros = jnp.zeros((L,), jnp.int32)
    tmask = lanes >= 0
    lane0 = lanes == 0
    lanebins = lanes * HSTRIDE

    def zero_hist(ncopies):
        @plsc.parallel_loop(0, (HSTRIDE * ncopies * L) // L, unroll=4)
        def _zb(i):
            hist_v[pl.ds(i * L, L)] = zeros

    def pass1_chunk(c0):
        @plsc.parallel_loop(0, CVREG // NHIST, unroll=4)
        def _p1(j):
            for t in range(NHIST):
                base = c0 + (j * NHIST + t) * L
                ikey = _tr(row_v[pl.ds(base, L)])
                digit = ((ikey >> 24) & 0xFF) ^ 0x80
                plsc.addupdate_scatter(
                    hist_v, [digit + (lanebins + t * HSIZE)], ones,
                    mask=tmask)

    def gather_keys(iv, valid):
        return _tr(plsc.load_gather(row_v, [iv], mask=valid))

    def hist_pass(M, shift, hi):
        zero_hist(1)
        def hb(j, _):
            base = j * L
            valid = (base + lanes) < M
            kv = gather_keys(idxs_v[pl.ds(base, L)], valid)
            digit = jnp.where(kv > hi, 255, (kv >> shift) & 0xFF)
            plsc.addupdate_scatter(hist_v, [digit + lanebins], ones,
                                   mask=valid)
            return 0
        lax.fori_loop(0, (M + L - 1) // L, hb, 0)

    def find_bin(ncopies):
        def fbv(i, carry):
            b, found, sab = carry
            v = L - 1 - i
            def lr(l, acc):
                a = acc
                for c in range(ncopies):
                    a = a + hist_v[pl.ds(l * HSTRIDE + c * HSIZE + v * L, L)]
                return a
            t_v = lax.fori_loop(0, L, lr, zeros, unroll=4)
            rc_v = lax.rev(plsc.cumsum(lax.rev(t_v, (0,))), (0,))
            need = K - sab
            cnt = jnp.sum((rc_v >= need).astype(jnp.int32))
            hit = cnt > 0
            b = jnp.where(found | (~hit), b, v * L + cnt - 1)
            return b, found | hit, sab + jnp.max(rc_v)
        b, _, _ = lax.fori_loop(
            0, L, fbv, (jnp.int32(-1), jnp.bool_(False), jnp.int32(0)))
        return b

    def split1(tp):
        @plsc.parallel_loop(0, NVREG, unroll=8, carry=zeros)
        def _sb(j, boff):
            base = j * L
            kv = _tr(row_v[pl.ds(base, L)])
            m = kv >= tp
            c = plsc.cumsum(m.astype(jnp.int32))
            plsc.store_scatter(idxs_v, [boff + c - 1], base + lanes, mask=m)
            return boff + plsc.all_reduce_population_count(m)
        return _sb

    def split(M, tp):
        def sb(j, boff):
            base = j * L
            valid = (base + lanes) < M
            iv = idxs_v[pl.ds(base, L)]
            kv = gather_keys(iv, valid)
            m = (kv >= tp) & valid
            c = plsc.cumsum(m.astype(jnp.int32))
            plsc.store_scatter(idxs_v, [boff + c - 1], iv, mask=m)
            return boff + plsc.all_reduce_population_count(m)
        return lax.fori_loop(0, (M + L - 1) // L, sb, zeros)

    def chunk_copy(row, c):
        return pltpu.make_async_copy(
            x_hbm.at[row, pl.ds(c * CWORDS, CWORDS)],
            row_v.at[pl.ds(c * CWORDS, CWORDS)], sems[c])

    def row_body(r, _):
        row = row0 + r
        for c in range(NCHUNK):
            chunk_copy(row, c).start()
        zero_hist(NHIST)
        for c in range(NCHUNK):
            chunk_copy(row, c).wait()
            pass1_chunk(c * CWORDS)

        b1 = find_bin(NHIST)
        tp = ((b1 ^ 0x80) & 0xFF) << 24
        boff = split1(tp)

        def level(lvl, carry):
            tp, M = carry
            shift = 16 - lvl * 8
            hi = tp | (lax.shift_left(jnp.int32(1), shift + 8) - 1)
            hist_pass(M, shift, hi)
            b = find_bin(1)
            tp = tp | lax.shift_left(b, shift)
            boff = split(M, tp)
            return tp, jnp.max(boff)
        tp, M = lax.fori_loop(0, 3, level, (tp, jnp.max(boff)))

        def fb(j, carry):
            doff, boff = carry
            base = j * L
            valid = (base + lanes) < M
            iv = idxs_v[pl.ds(base, L)]
            kv = gather_keys(iv, valid)
            m_gt = (kv > tp) & valid
            m_eq = (kv == tp) & valid
            c_gt = plsc.cumsum(m_gt.astype(jnp.int32))
            plsc.store_scatter(didx_v, [doff + c_gt - 1], iv, mask=m_gt)
            c_eq = plsc.cumsum(m_eq.astype(jnp.int32))
            plsc.store_scatter(idxs_v, [boff + c_eq - 1], iv, mask=m_eq)
            doff = doff + plsc.all_reduce_population_count(m_gt)
            boff = boff + plsc.all_reduce_population_count(m_eq)
            return doff, boff
        doff, _ = lax.fori_loop(0, (M + L - 1) // L, fb, (zeros, zeros))
        kneed = K - jnp.max(doff)
        doffs = jnp.max(doff)

        def ap(j, _):
            pos = j * L + lanes
            m = pos < kneed
            iv = idxs_v[pl.ds(j * L, L)]
            plsc.store_scatter(didx_v, [doffs + pos], iv, mask=m)
            return 0
        lax.fori_loop(0, (kneed + L - 1) // L, ap, 0)

        Is = [didx_v[pl.ds(a * L, L)] for a in range(K // L)]
        Ks = [gather_keys(iv, tmask) for iv in Is]
        rsplat = jnp.full((L,), r, jnp.int32)

        @plsc.parallel_loop(0, K, unroll=2)
        def _rk(c):
            csplat = jnp.full((L,), c, jnp.int32)
            ic = plsc.load_gather(didx_v, [csplat])
            kc = gather_keys(ic, tmask)
            rank = zeros
            for a in range(K // L):
                m = (Ks[a] > kc) | ((Ks[a] == kc) & (Is[a] < ic))
                rank = rank + plsc.all_reduce_population_count(m)
            plsc.store_scatter(ostage_v, [rsplat, rank], ic, mask=lane0)
        return 0

    lax.fori_loop(0, ROWS_PER_W, row_body, 0)
    pltpu.make_async_copy(ostage_v, out_hbm.at[pl.ds(row0, ROWS_PER_W)],
                          sem_out).start()
    pltpu.make_async_copy(ostage_v, out_hbm.at[pl.ds(row0, ROWS_PER_W)],
                          sem_out).wait()


@jax.jit
def kernel(input_tensor):
    x_i32 = lax.bitcast_convert_type(input_tensor, jnp.int32)
    mesh = plsc.VectorSubcoreMesh(core_axis_name="c", subcore_axis_name="s",
                                  num_cores=NC, num_subcores=NS)
    f = pl.kernel(
        _body,
        out_type=jax.ShapeDtypeStruct((NROWS, K), jnp.int32),
        mesh=mesh,
        scratch_types=[
            pltpu.VMEM((NCOLS,), jnp.int32),
            pltpu.VMEM((NCOLS,), jnp.int32),
            pltpu.VMEM((HSIZE * NHIST,), jnp.int32),
            pltpu.VMEM((K,), jnp.int32),
            pltpu.VMEM((ROWS_PER_W, K), jnp.int32),
            [pltpu.SemaphoreType.DMA] * NCHUNK,
            pltpu.SemaphoreType.DMA,
        ],
        compiler_params=pltpu.CompilerParams(
            needs_layout_passes=False,
            use_tc_tiling_on_sc=True,
        ),
    )
    return f(x_i32)


# speedup vs baseline: 1.1892x; 1.0124x over previous
import jax
import jax.numpy as jnp
from jax import lax
from jax.experimental import pallas as pl
from jax.experimental.pallas import tpu as pltpu
from jax.experimental.pallas import tpu_sc as plsc

NROWS = 128
NCOLS = 32768
K = 64
NC, NS, L = 2, 16, 16
NW = NC * NS
ROWS_PER_W = NROWS // NW
NVREG = NCOLS // L
NBINS = 256
HSTRIDE = NBINS + 1
HSIZE = HSTRIDE * L
NHIST = 4
NCHUNK = 4
CWORDS = NCOLS // NCHUNK
CVREG = CWORDS // L


def _tr(k):
    return jnp.where(k < 0, k ^ jnp.int32(0x7FFFFFFF), k)


def _body(x_hbm, out_hbm, row_v, idxs_v, hist_v, didx_v, ostage_v, sems,
          sem_out):
    wid = lax.axis_index("s") * NC + lax.axis_index("c")
    row0 = wid * ROWS_PER_W
    lanes = lax.iota(jnp.int32, L)
    ones = jnp.ones((L,), jnp.int32)
    zeros = jnp.zeros((L,), jnp.int32)
    tmask = lanes >= 0
    lane0 = lanes == 0
    lanebins = lanes * HSTRIDE

    def zero_hist(ncopies):
        @plsc.parallel_loop(0, (HSTRIDE * ncopies * L) // L, unroll=4)
        def _zb(i):
            hist_v[pl.ds(i * L, L)] = zeros

    def pass1_chunk(c0):
        @plsc.parallel_loop(0, CVREG // NHIST, unroll=4)
        def _p1(j):
            for t in range(NHIST):
                base = c0 + (j * NHIST + t) * L
                ikey = _tr(row_v[pl.ds(base, L)])
                digit = ((ikey >> 24) & 0xFF) ^ 0x80
                plsc.addupdate_scatter(
                    hist_v, [digit + (lanebins + t * HSIZE)], ones,
                    mask=tmask)

    def gather_keys(iv, valid):
        return _tr(plsc.load_gather(row_v, [iv], mask=valid))

    def hist_pass(M, shift, hi):
        zero_hist(1)
        def hb(j, _):
            base = j * L
            valid = (base + lanes) < M
            kv = gather_keys(idxs_v[pl.ds(base, L)], valid)
            digit = jnp.where(kv > hi, 255, (kv >> shift) & 0xFF)
            plsc.addupdate_scatter(hist_v, [digit + lanebins], ones,
                                   mask=valid)
            return 0
        lax.fori_loop(0, (M + L - 1) // L, hb, 0)

    def find_bin(ncopies):
        def fbv(i, carry):
            b, found, sab = carry
            v = L - 1 - i
            def lr(l, acc):
                a = acc
                for c in range(ncopies):
                    a = a + hist_v[pl.ds(l * HSTRIDE + c * HSIZE + v * L, L)]
                return a
            t_v = lax.fori_loop(0, L, lr, zeros, unroll=4)
            rc_v = lax.rev(plsc.cumsum(lax.rev(t_v, (0,))), (0,))
            need = K - sab
            cnt = jnp.sum((rc_v >= need).astype(jnp.int32))
            hit = cnt > 0
            b = jnp.where(found | (~hit), b, v * L + cnt - 1)
            return b, found | hit, sab + jnp.max(rc_v)
        b, _, _ = lax.fori_loop(
            0, L, fbv, (jnp.int32(-1), jnp.bool_(False), jnp.int32(0)))
        return b

    def split1(tp):
        @plsc.parallel_loop(0, NVREG, unroll=8, carry=zeros)
        def _sb(j, boff):
            base = j * L
            kv = _tr(row_v[pl.ds(base, L)])
            m = kv >= tp
            c = plsc.cumsum(m.astype(jnp.int32))
            plsc.store_scatter(idxs_v, [boff + c - 1], base + lanes, mask=m)
            return boff + plsc.all_reduce_population_count(m)
        return _sb

    def split(M, tp):
        def sb(j, boff):
            base = j * L
            valid = (base + lanes) < M
            iv = idxs_v[pl.ds(base, L)]
            kv = gather_keys(iv, valid)
            m = (kv >= tp) & valid
            c = plsc.cumsum(m.astype(jnp.int32))
            plsc.store_scatter(idxs_v, [boff + c - 1], iv, mask=m)
            return boff + plsc.all_reduce_population_count(m)
        return lax.fori_loop(0, (M + L - 1) // L, sb, zeros)

    def chunk_copy(row, c):
        return pltpu.make_async_copy(
            x_hbm.at[row, pl.ds(c * CWORDS, CWORDS)],
            row_v.at[pl.ds(c * CWORDS, CWORDS)], sems[c])

    def row_body(r, _):
        row = row0 + r
        for c in range(NCHUNK):
            chunk_copy(row, c).start()
        zero_hist(NHIST)
        for c in range(NCHUNK):
            chunk_copy(row, c).wait()
            pass1_chunk(c * CWORDS)

        b1 = find_bin(NHIST)
        tp = ((b1 ^ 0x80) & 0xFF) << 24
        boff = split1(tp)

        def level(lvl, carry):
            tp, M = carry
            shift = 16 - lvl * 8
            hi = tp | (lax.shift_left(jnp.int32(1), shift + 8) - 1)
            hist_pass(M, shift, hi)
            b = find_bin(1)
            tp = tp | lax.shift_left(b, shift)
            boff = split(M, tp)
            return tp, jnp.max(boff)
        tp, M = lax.fori_loop(0, 3, level, (tp, jnp.max(boff)))

        def fb(j, carry):
            doff, boff = carry
            base = j * L
            valid = (base + lanes) < M
            iv = idxs_v[pl.ds(base, L)]
            kv = gather_keys(iv, valid)
            m_gt = (kv > tp) & valid
            m_eq = (kv == tp) & valid
            c_gt = plsc.cumsum(m_gt.astype(jnp.int32))
            plsc.store_scatter(didx_v, [doff + c_gt - 1], iv, mask=m_gt)
            c_eq = plsc.cumsum(m_eq.astype(jnp.int32))
            plsc.store_scatter(idxs_v, [boff + c_eq - 1], iv, mask=m_eq)
            doff = doff + plsc.all_reduce_population_count(m_gt)
            boff = boff + plsc.all_reduce_population_count(m_eq)
            return doff, boff
        doff, _ = lax.fori_loop(0, (M + L - 1) // L, fb, (zeros, zeros))
        kneed = K - jnp.max(doff)
        doffs = jnp.max(doff)

        def ap(j, _):
            pos = j * L + lanes
            m = pos < kneed
            iv = idxs_v[pl.ds(j * L, L)]
            plsc.store_scatter(didx_v, [doffs + pos], iv, mask=m)
            return 0
        lax.fori_loop(0, (kneed + L - 1) // L, ap, 0)

        Is = [didx_v[pl.ds(a * L, L)] for a in range(K // L)]
        Ks = [gather_keys(iv, tmask) for iv in Is]
        rsplat = jnp.full((L,), r, jnp.int32)

        @plsc.parallel_loop(0, K, unroll=2)
        def _rk(c):
            csplat = jnp.full((L,), c, jnp.int32)
            ic = plsc.load_gather(didx_v, [csplat])
            kc = gather_keys(ic, tmask)
            rank = zeros
            for a in range(K // L):
                m = (Ks[a] > kc) | ((Ks[a] == kc) & (Is[a] < ic))
                rank = rank + plsc.all_reduce_population_count(m)
            plsc.store_scatter(ostage_v, [rsplat, rank], ic, mask=lane0)
        return 0

    lax.fori_loop(0, ROWS_PER_W, row_body, 0)
    pltpu.make_async_copy(ostage_v, out_hbm.at[pl.ds(row0, ROWS_PER_W)],
                          sem_out).start()
    pltpu.make_async_copy(ostage_v, out_hbm.at[pl.ds(row0, ROWS_PER_W)],
                          sem_out).wait()


@jax.jit
def kernel(input_tensor):
    x_i32 = lax.bitcast_convert_type(input_tensor, jnp.int32)
    mesh = plsc.VectorSubcoreMesh(core_axis_name="c", subcore_axis_name="s",
                                  num_cores=NC, num_subcores=NS)
    f = pl.kernel(
        _body,
        out_type=jax.ShapeDtypeStruct((NROWS, K), jnp.int32),
        mesh=mesh,
        scratch_types=[
            pltpu.VMEM((NCOLS,), jnp.int32),
            pltpu.VMEM((NCOLS,), jnp.int32),
            pltpu.VMEM((HSIZE * NHIST,), jnp.int32),
            pltpu.VMEM((K,), jnp.int32),
            pltpu.VMEM((ROWS_PER_W, K), jnp.int32),
            [pltpu.SemaphoreType.DMA] * NCHUNK,
            pltpu.SemaphoreType.DMA,
        ],
        compiler_params=pltpu.CompilerParams(
            needs_layout_passes=False,
            use_tc_tiling_on_sc=True,
        ),
    )
    return f(x_i32)
